# Initial kernel scaffold; baseline (speedup 1.0000x reference)
import jax
import jax.numpy as jnp
from jax import lax
from jax.experimental import pallas as pl
from jax.experimental.pallas import tpu as pltpu
from jax.experimental.pallas import tpu_sc as plsc

NC = 2
NS = 16
NW = NC * NS
LANES = 16

N = 10000
E = 320000
D = 128
R = 3
NPAD = 10240
CNT = R * NPAD
CNT_ROWS = CNT // 128
CH = E // NW
EB = 80
NB = CH // EB

_mesh = plsc.VectorSubcoreMesh(core_axis_name="c", subcore_axis_name="s")


def _wid():
    return lax.axis_index("s") * NC + lax.axis_index("c")


def _preproc_body(src_hbm, dst_hbm, attr_hbm,
                  gidx_hbm, cidx_hbm, cnt_hbm,
                  sv, dv, av, gv, cv, ones_v, ix_v, zb_v, shared_cnt):
    c = lax.axis_index("c")
    s = lax.axis_index("s")
    wid = _wid()
    base = wid * CH

    pltpu.sync_copy(src_hbm.at[pl.ds(base, CH)], sv)
    pltpu.sync_copy(dst_hbm.at[pl.ds(base, CH)], dv)
    pltpu.sync_copy(attr_hbm.at[pl.ds(base, CH)], av)

    def zb_body(i, _):
        zb_v[pl.ds(i * LANES, LANES)] = jnp.zeros((LANES,), jnp.float32)
        return 0
    lax.fori_loop(0, (CNT // NS) // LANES, zb_body, 0)

    for k in range(EB // LANES):
        ones_v[pl.ds(k * LANES, LANES)] = jnp.ones((LANES,), jnp.float32)

    def idx_body(i, _):
        sl = pl.ds(i * LANES, LANES)
        a = av[sl]
        gv[sl] = a * N + sv[sl]
        cv[sl] = a * NPAD + dv[sl]
        return 0
    lax.fori_loop(0, CH // LANES, idx_body, 0)

    pltpu.sync_copy(gv, gidx_hbm.at[pl.ds(base, CH)])
    pltpu.sync_copy(cv, cidx_hbm.at[pl.ds(base, CH)])

    pltpu.sync_copy(zb_v, shared_cnt.at[pl.ds(s * (CNT // NS), CNT // NS)])
    plsc.subcore_barrier()

    def cnt_body(b, _):
        off = b * EB
        for k in range(EB // LANES):
            sl = pl.ds(k * LANES, LANES)
            ix_v[sl] = cv[pl.ds(off + k * LANES, LANES)]
        pltpu.sync_copy(ones_v, shared_cnt.at[ix_v], add=True)
        return 0
    lax.fori_loop(0, NB, cnt_body, 0)

    plsc.subcore_barrier()
    sl = pl.ds(s * (CNT // NS), CNT // NS)
    pltpu.sync_copy(shared_cnt.at[sl], cnt_hbm.at[c, sl])


_preproc = pl.kernel(
    _preproc_body,
    out_type=(
        jax.ShapeDtypeStruct((E,), jnp.int32),
        jax.ShapeDtypeStruct((E,), jnp.int32),
        jax.ShapeDtypeStruct((NC, CNT), jnp.float32),
    ),
    mesh=_mesh,
    scratch_types=[
        pltpu.VMEM((CH,), jnp.int32),
        pltpu.VMEM((CH,), jnp.int32),
        pltpu.VMEM((CH,), jnp.int32),
        pltpu.VMEM((CH,), jnp.int32),
        pltpu.VMEM((CH,), jnp.int32),
        pltpu.VMEM((EB,), jnp.float32),
        pltpu.VMEM((EB,), jnp.int32),
        pltpu.VMEM((CNT // NS,), jnp.float32),
        pltpu.VMEM_SHARED((CNT,), jnp.float32),
    ],
)


def _edge_body(y_hbm, inv_hbm, gidx_hbm, cidx_hbm, dst_hbm,
               part_hbm,
               gix_v, cix_v, dix_v, s_v, msgs, zb_v, shared_acc):
    c = lax.axis_index("c")
    s = lax.axis_index("s")
    wid = _wid()
    rows_per_tile = N // NS
    zrows = rows_per_tile // 5

    def zb_body(i, _):
        for k in range(D // LANES):
            zb_v[i, pl.ds(k * LANES, LANES)] = jnp.zeros((LANES,), jnp.float32)
        return 0
    lax.fori_loop(0, zrows, zb_body, 0)
    for t in range(5):
        pltpu.sync_copy(
            zb_v, shared_acc.at[pl.ds(s * rows_per_tile + t * zrows, zrows)])
    plsc.subcore_barrier()

    def batch_body(b, _):
        base = wid * CH + b * EB
        pltpu.sync_copy(gidx_hbm.at[pl.ds(base, EB)], gix_v)
        pltpu.sync_copy(cidx_hbm.at[pl.ds(base, EB)], cix_v)
        pltpu.sync_copy(dst_hbm.at[pl.ds(base, EB)], dix_v)
        pltpu.sync_copy(y_hbm.at[gix_v], msgs)
        pltpu.sync_copy(inv_hbm.at[cix_v], s_v)

        def row_body(j, _):
            sv = s_v[j]
            for k in range(D // LANES):
                sl = pl.ds(k * LANES, LANES)
                msgs[j, sl] = msgs[j, sl] * sv
            return 0
        lax.fori_loop(0, EB, row_body, 0)

        pltpu.sync_copy(msgs, shared_acc.at[dix_v], add=True)
        return 0
    lax.fori_loop(0, NB, batch_body, 0)

    plsc.subcore_barrier()
    sl = pl.ds(s * rows_per_tile, rows_per_tile)
    pltpu.sync_copy(shared_acc.at[sl], part_hbm.at[c, sl])


_edge = pl.kernel(
    _edge_body,
    out_type=jax.ShapeDtypeStruct((NC, N, D), jnp.float32),
    mesh=_mesh,
    scratch_types=[
        pltpu.VMEM((EB,), jnp.int32),
        pltpu.VMEM((EB,), jnp.int32),
        pltpu.VMEM((EB,), jnp.int32),
        pltpu.VMEM((EB,), jnp.float32),
        pltpu.VMEM((EB, D), jnp.float32),
        pltpu.VMEM((N // NS // 5, D), jnp.float32),
        pltpu.VMEM_SHARED((N, D), jnp.float32),
    ],
)


def _inv_body(cnt_ref, inv_ref):
    inv_ref[...] = 1.0 / jnp.maximum(cnt_ref[0] + cnt_ref[1], 1.0)


def _tc_inv(cnt_part):
    return pl.pallas_call(
        _inv_body,
        out_shape=jax.ShapeDtypeStruct((CNT_ROWS, 128), jnp.float32),
    )(cnt_part.reshape(NC, CNT_ROWS, 128))


_BN = 1000


def _mm1_body(x_ref, wrel_ref, wroot_ref, b_ref, y_ref, root_ref):
    xb = x_ref[...]
    root_ref[...] = jnp.dot(xb, wroot_ref[...],
                            preferred_element_type=jnp.float32) + b_ref[0]
    for r in range(R):
        y_ref[r] = jnp.dot(xb, wrel_ref[r], preferred_element_type=jnp.float32)


def _tc_mm1(x, w_rel, w_root, b):
    return pl.pallas_call(
        _mm1_body,
        grid=(N // _BN,),
        in_specs=[
            pl.BlockSpec((_BN, D), lambda i: (i, 0)),
            pl.BlockSpec((R, D, D), lambda i: (0, 0, 0)),
            pl.BlockSpec((D, D), lambda i: (0, 0)),
            pl.BlockSpec((1, D), lambda i: (0, 0)),
        ],
        out_specs=[
            pl.BlockSpec((R, _BN, D), lambda i: (0, i, 0)),
            pl.BlockSpec((_BN, D), lambda i: (i, 0)),
        ],
        out_shape=[
            jax.ShapeDtypeStruct((R, N, D), jnp.float32),
            jax.ShapeDtypeStruct((N, D), jnp.float32),
        ],
    )(x, w_rel, w_root, b.reshape(1, D))


def _mm2_body(part_ref, root1_ref, wrel_ref, wroot_ref, b_ref, y_ref, root_ref):
    hb = jnp.maximum(part_ref[0] + part_ref[1] + root1_ref[...], 0.0)
    root_ref[...] = jnp.dot(hb, wroot_ref[...],
                            preferred_element_type=jnp.float32) + b_ref[0]
    for r in range(R):
        y_ref[r] = jnp.dot(hb, wrel_ref[r], preferred_element_type=jnp.float32)


def _tc_mm2(part, root1, w_rel, w_root, b):
    return pl.pallas_call(
        _mm2_body,
        grid=(N // _BN,),
        in_specs=[
            pl.BlockSpec((NC, _BN, D), lambda i: (0, i, 0)),
            pl.BlockSpec((_BN, D), lambda i: (i, 0)),
            pl.BlockSpec((R, D, D), lambda i: (0, 0, 0)),
            pl.BlockSpec((D, D), lambda i: (0, 0)),
            pl.BlockSpec((1, D), lambda i: (0, 0)),
        ],
        out_specs=[
            pl.BlockSpec((R, _BN, D), lambda i: (0, i, 0)),
            pl.BlockSpec((_BN, D), lambda i: (i, 0)),
        ],
        out_shape=[
            jax.ShapeDtypeStruct((R, N, D), jnp.float32),
            jax.ShapeDtypeStruct((N, D), jnp.float32),
        ],
    )(part, root1, w_rel, w_root, b.reshape(1, D))


def _final_body(part_ref, root_ref, out_ref):
    out_ref[...] = part_ref[0] + part_ref[1] + root_ref[...]


def _tc_final(part, root):
    return pl.pallas_call(
        _final_body,
        grid=(N // _BN,),
        in_specs=[
            pl.BlockSpec((NC, _BN, D), lambda i: (0, i, 0)),
            pl.BlockSpec((_BN, D), lambda i: (i, 0)),
        ],
        out_specs=pl.BlockSpec((_BN, D), lambda i: (i, 0)),
        out_shape=jax.ShapeDtypeStruct((N, D), jnp.float32),
    )(part, root)


def kernel(x, edge_index, edge_attr, w_rel1, w_root1, b1, w_rel2, w_root2, b2):
    src = edge_index[0]
    dst = edge_index[1]

    gidx, cidx, cnt_part = _preproc(src, dst, edge_attr)
    inv1d = _tc_inv(cnt_part).reshape(CNT)

    y1, root1 = _tc_mm1(x, w_rel1, w_root1, b1)
    part1 = _edge(y1.reshape(R * N, D), inv1d, gidx, cidx, dst)

    y2, root2 = _tc_mm2(part1, root1, w_rel2, w_root2, b2)
    part2 = _edge(y2.reshape(R * N, D), inv1d, gidx, cidx, dst)

    return _tc_final(part2, root2)


# speedup vs baseline: 6.9136x; 6.9136x over previous
import jax
import jax.numpy as jnp
from jax import lax
from jax.experimental import pallas as pl
from jax.experimental.pallas import tpu as pltpu
from jax.experimental.pallas import tpu_sc as plsc

NC = 2
NS = 16
NW = NC * NS
LANES = 16

N = 10000
E = 320000
D = 128
R = 3
NPAD = 10240
N2 = 10240
CNT = R * NPAD
CNT_ROWS = CNT // 128
CH = E // NW
EB = 80
NB = CH // EB

_mesh = plsc.VectorSubcoreMesh(core_axis_name="c", subcore_axis_name="s")


def _wid():
    return lax.axis_index("s") * NC + lax.axis_index("c")


def _preproc_body(src_hbm, dst_hbm, attr_hbm,
                  gidx_hbm, cidx_hbm, cnt_hbm,
                  sv, dv, av, gv, cv, ones_v, ix_v, zb_v, shared_cnt):
    c = lax.axis_index("c")
    s = lax.axis_index("s")
    wid = _wid()
    base = wid * CH

    pltpu.sync_copy(src_hbm.at[pl.ds(base, CH)], sv)
    pltpu.sync_copy(dst_hbm.at[pl.ds(base, CH)], dv)
    pltpu.sync_copy(attr_hbm.at[pl.ds(base, CH)], av)

    def zb_body(i, _):
        zb_v[pl.ds(i * LANES, LANES)] = jnp.zeros((LANES,), jnp.float32)
        return 0
    lax.fori_loop(0, (CNT // NS) // LANES, zb_body, 0)

    for k in range(EB // LANES):
        ones_v[pl.ds(k * LANES, LANES)] = jnp.ones((LANES,), jnp.float32)

    def idx_body(i, _):
        sl = pl.ds(i * LANES, LANES)
        a = av[sl]
        gv[sl] = a * N + sv[sl]
        cv[sl] = a * NPAD + dv[sl]
        return 0
    lax.fori_loop(0, CH // LANES, idx_body, 0)

    pltpu.sync_copy(gv, gidx_hbm.at[pl.ds(base, CH)])
    pltpu.sync_copy(cv, cidx_hbm.at[pl.ds(base, CH)])

    pltpu.sync_copy(zb_v, shared_cnt.at[pl.ds(s * (CNT // NS), CNT // NS)])
    plsc.subcore_barrier()

    def cnt_body(b, _):
        off = b * EB
        for k in range(EB // LANES):
            sl = pl.ds(k * LANES, LANES)
            ix_v[sl] = cv[pl.ds(off + k * LANES, LANES)]
        pltpu.sync_copy(ones_v, shared_cnt.at[ix_v], add=True)
        return 0
    lax.fori_loop(0, NB, cnt_body, 0)

    plsc.subcore_barrier()
    sl = pl.ds(s * (CNT // NS), CNT // NS)
    pltpu.sync_copy(shared_cnt.at[sl],
                    cnt_hbm.at[pl.ds(c * CNT + s * (CNT // NS), CNT // NS)])


_preproc = pl.kernel(
    _preproc_body,
    out_type=(
        jax.ShapeDtypeStruct((E,), jnp.int32),
        jax.ShapeDtypeStruct((E,), jnp.int32),
        jax.ShapeDtypeStruct((NC * CNT,), jnp.float32),
    ),
    mesh=_mesh,
    scratch_types=[
        pltpu.VMEM((CH,), jnp.int32),
        pltpu.VMEM((CH,), jnp.int32),
        pltpu.VMEM((CH,), jnp.int32),
        pltpu.VMEM((CH,), jnp.int32),
        pltpu.VMEM((CH,), jnp.int32),
        pltpu.VMEM((EB,), jnp.float32),
        pltpu.VMEM((EB,), jnp.int32),
        pltpu.VMEM((CNT // NS,), jnp.float32),
        pltpu.VMEM_SHARED((CNT,), jnp.float32),
    ],
)


def _edge_body(y_hbm, inv_hbm, gidx_hbm, cidx_hbm, dst_hbm,
               part_hbm,
               gix_v, cix_v, dix_v, s_v, msgs, zb_v, shared_acc):
    c = lax.axis_index("c")
    s = lax.axis_index("s")
    wid = _wid()
    rows_per_tile = N2 // NS
    zrows = rows_per_tile // 5

    def zb_body(i, _):
        for k in range(D // LANES):
            zb_v[i, pl.ds(k * LANES, LANES)] = jnp.zeros((LANES,), jnp.float32)
        return 0
    lax.fori_loop(0, zrows, zb_body, 0)
    for t in range(5):
        pltpu.sync_copy(
            zb_v, shared_acc.at[pl.ds(s * rows_per_tile + t * zrows, zrows)])
    plsc.subcore_barrier()

    def batch_body(b, _):
        base = wid * CH + b * EB
        pltpu.sync_copy(gidx_hbm.at[pl.ds(base, EB)], gix_v)
        pltpu.sync_copy(cidx_hbm.at[pl.ds(base, EB)], cix_v)
        pltpu.sync_copy(dst_hbm.at[pl.ds(base, EB)], dix_v)
        pltpu.sync_copy(y_hbm.at[gix_v], msgs)
        pltpu.sync_copy(inv_hbm.at[cix_v], s_v)

        def grp_body(g, _):
            sg = s_v[pl.ds(g * LANES, LANES)]
            for l in range(LANES):
                sv = sg[l]
                row = g * LANES + l
                for k in range(D // LANES):
                    sl = pl.ds(k * LANES, LANES)
                    msgs[row, sl] = msgs[row, sl] * sv
            return 0
        lax.fori_loop(0, EB // LANES, grp_body, 0)

        pltpu.sync_copy(msgs, shared_acc.at[dix_v], add=True)
        return 0
    lax.fori_loop(0, NB, batch_body, 0)

    plsc.subcore_barrier()
    sl = pl.ds(s * rows_per_tile, rows_per_tile)
    pltpu.sync_copy(shared_acc.at[sl], part_hbm.at[c, sl])


_edge = pl.kernel(
    _edge_body,
    out_type=jax.ShapeDtypeStruct((NC, N2, D), jnp.float32),
    mesh=_mesh,
    scratch_types=[
        pltpu.VMEM((EB,), jnp.int32),
        pltpu.VMEM((EB,), jnp.int32),
        pltpu.VMEM((EB,), jnp.int32),
        pltpu.VMEM((EB,), jnp.float32),
        pltpu.VMEM((EB, D), jnp.float32),
        pltpu.VMEM((N2 // NS // 5, D), jnp.float32),
        pltpu.VMEM_SHARED((N2, D), jnp.float32),
    ],
)


def _inv_body(cnt_ref, inv_ref):
    inv_ref[...] = 1.0 / jnp.maximum(cnt_ref[0] + cnt_ref[1], 1.0)


def _tc_inv(cnt_part):
    return pl.pallas_call(
        _inv_body,
        out_shape=jax.ShapeDtypeStruct((CNT_ROWS, 128), jnp.float32),
    )(cnt_part.reshape(NC, CNT_ROWS, 128))


_BN = 1000


def _mm1_body(x_ref, wrel_ref, wroot_ref, b_ref, y_ref, root_ref):
    xb = x_ref[...]
    root_ref[...] = jnp.dot(xb, wroot_ref[...],
                            preferred_element_type=jnp.float32) + b_ref[0]
    for r in range(R):
        y_ref[r] = jnp.dot(xb, wrel_ref[r], preferred_element_type=jnp.float32)


def _tc_mm1(x, w_rel, w_root, b):
    return pl.pallas_call(
        _mm1_body,
        grid=(N // _BN,),
        in_specs=[
            pl.BlockSpec((_BN, D), lambda i: (i, 0)),
            pl.BlockSpec((R, D, D), lambda i: (0, 0, 0)),
            pl.BlockSpec((D, D), lambda i: (0, 0)),
            pl.BlockSpec((1, D), lambda i: (0, 0)),
        ],
        out_specs=[
            pl.BlockSpec((R, _BN, D), lambda i: (0, i, 0)),
            pl.BlockSpec((_BN, D), lambda i: (i, 0)),
        ],
        out_shape=[
            jax.ShapeDtypeStruct((R, N, D), jnp.float32),
            jax.ShapeDtypeStruct((N, D), jnp.float32),
        ],
    )(x, w_rel, w_root, b.reshape(1, D))


def _mm2_body(part_ref, root1_ref, wrel_ref, wroot_ref, b_ref, y_ref, root_ref):
    hb = jnp.maximum(part_ref[0] + part_ref[1] + root1_ref[...], 0.0)
    root_ref[...] = jnp.dot(hb, wroot_ref[...],
                            preferred_element_type=jnp.float32) + b_ref[0]
    for r in range(R):
        y_ref[r] = jnp.dot(hb, wrel_ref[r], preferred_element_type=jnp.float32)


def _tc_mm2(part, root1, w_rel, w_root, b):
    return pl.pallas_call(
        _mm2_body,
        grid=(N // _BN,),
        in_specs=[
            pl.BlockSpec((NC, _BN, D), lambda i: (0, i, 0)),
            pl.BlockSpec((_BN, D), lambda i: (i, 0)),
            pl.BlockSpec((R, D, D), lambda i: (0, 0, 0)),
            pl.BlockSpec((D, D), lambda i: (0, 0)),
            pl.BlockSpec((1, D), lambda i: (0, 0)),
        ],
        out_specs=[
            pl.BlockSpec((R, _BN, D), lambda i: (0, i, 0)),
            pl.BlockSpec((_BN, D), lambda i: (i, 0)),
        ],
        out_shape=[
            jax.ShapeDtypeStruct((R, N, D), jnp.float32),
            jax.ShapeDtypeStruct((N, D), jnp.float32),
        ],
    )(part, root1, w_rel, w_root, b.reshape(1, D))


def _final_body(part_ref, root_ref, out_ref):
    out_ref[...] = part_ref[0] + part_ref[1] + root_ref[...]


def _tc_final(part, root):
    return pl.pallas_call(
        _final_body,
        grid=(N // _BN,),
        in_specs=[
            pl.BlockSpec((NC, _BN, D), lambda i: (0, i, 0)),
            pl.BlockSpec((_BN, D), lambda i: (i, 0)),
        ],
        out_specs=pl.BlockSpec((_BN, D), lambda i: (i, 0)),
        out_shape=jax.ShapeDtypeStruct((N, D), jnp.float32),
    )(part, root)


def kernel(x, edge_index, edge_attr, w_rel1, w_root1, b1, w_rel2, w_root2, b2):
    src = edge_index[0]
    dst = edge_index[1]

    gidx, cidx, cnt_part = _preproc(src, dst, edge_attr)
    inv1d = _tc_inv(cnt_part).reshape(CNT)

    y1, root1 = _tc_mm1(x, w_rel1, w_root1, b1)
    part1 = _edge(y1.reshape(R * N, D), inv1d, gidx, cidx, dst)

    y2, root2 = _tc_mm2(part1, root1, w_rel2, w_root2, b2)
    part2 = _edge(y2.reshape(R * N, D), inv1d, gidx, cidx, dst)

    return _tc_final(part2, root2)


# speedup vs baseline: 13.3837x; 1.9358x over previous
import jax
import jax.numpy as jnp
from jax import lax
from jax.experimental import pallas as pl
from jax.experimental.pallas import tpu as pltpu
from jax.experimental.pallas import tpu_sc as plsc

NC = 2
NS = 16
NW = NC * NS
LANES = 16

N = 10000
E = 320000
D = 128
R = 3
NPAD = 10240
N2 = 10240
CNT = R * NPAD
CNT_ROWS = CNT // 128
CH = E // NW
PB = 80
PNB = CH // PB

_mesh = plsc.VectorSubcoreMesh(core_axis_name="c", subcore_axis_name="s")


def _wid():
    return lax.axis_index("s") * NC + lax.axis_index("c")


def _preproc_body(src_hbm, dst_hbm, attr_hbm,
                  gidx_hbm, cidx_hbm, cnt_hbm,
                  sv, dv, av, gv, cv, ones_v, ix_v, zb_v, shared_cnt):
    c = lax.axis_index("c")
    s = lax.axis_index("s")
    wid = _wid()
    base = wid * CH

    pltpu.sync_copy(src_hbm.at[pl.ds(base, CH)], sv)
    pltpu.sync_copy(dst_hbm.at[pl.ds(base, CH)], dv)
    pltpu.sync_copy(attr_hbm.at[pl.ds(base, CH)], av)

    def zb_body(i, _):
        zb_v[pl.ds(i * LANES, LANES)] = jnp.zeros((LANES,), jnp.float32)
        return 0
    lax.fori_loop(0, (CNT // NS) // LANES, zb_body, 0)

    for k in range(PB // LANES):
        ones_v[pl.ds(k * LANES, LANES)] = jnp.ones((LANES,), jnp.float32)

    def idx_body(i, _):
        sl = pl.ds(i * LANES, LANES)
        a = av[sl]
        gv[sl] = a * N + sv[sl]
        cv[sl] = a * NPAD + dv[sl]
        return 0
    lax.fori_loop(0, CH // LANES, idx_body, 0)

    pltpu.sync_copy(gv, gidx_hbm.at[pl.ds(base, CH)])
    pltpu.sync_copy(cv, cidx_hbm.at[pl.ds(base, CH)])

    pltpu.sync_copy(zb_v, shared_cnt.at[pl.ds(s * (CNT // NS), CNT // NS)])
    plsc.subcore_barrier()

    def cnt_body(b, _):
        off = b * PB
        for k in range(PB // LANES):
            sl = pl.ds(k * LANES, LANES)
            ix_v[sl] = cv[pl.ds(off + k * LANES, LANES)]
        pltpu.sync_copy(ones_v, shared_cnt.at[ix_v], add=True)
        return 0
    lax.fori_loop(0, PNB, cnt_body, 0)

    plsc.subcore_barrier()
    sl = pl.ds(s * (CNT // NS), CNT // NS)
    pltpu.sync_copy(shared_cnt.at[sl],
                    cnt_hbm.at[pl.ds(c * CNT + s * (CNT // NS), CNT // NS)])


_preproc = pl.kernel(
    _preproc_body,
    out_type=(
        jax.ShapeDtypeStruct((E,), jnp.int32),
        jax.ShapeDtypeStruct((E,), jnp.int32),
        jax.ShapeDtypeStruct((NC * CNT,), jnp.float32),
    ),
    mesh=_mesh,
    scratch_types=[
        pltpu.VMEM((CH,), jnp.int32),
        pltpu.VMEM((CH,), jnp.int32),
        pltpu.VMEM((CH,), jnp.int32),
        pltpu.VMEM((CH,), jnp.int32),
        pltpu.VMEM((CH,), jnp.int32),
        pltpu.VMEM((PB,), jnp.float32),
        pltpu.VMEM((PB,), jnp.int32),
        pltpu.VMEM((CNT // NS,), jnp.float32),
        pltpu.VMEM_SHARED((CNT,), jnp.float32),
    ],
)


SB = 80


def _scale_body(cidx_hbm, inv_hbm, scale_hbm, cv, sbig):
    wid = _wid()
    base = wid * CH
    pltpu.sync_copy(cidx_hbm.at[pl.ds(base, CH)], cv)

    def b_body(b, _):
        sl = pl.ds(b * SB, SB)
        pltpu.sync_copy(inv_hbm.at[cv.at[sl]], sbig.at[sl])
        return 0
    lax.fori_loop(0, CH // SB, b_body, 0)
    pltpu.sync_copy(sbig, scale_hbm.at[pl.ds(base, CH)])


_scale = pl.kernel(
    _scale_body,
    out_type=jax.ShapeDtypeStruct((E,), jnp.float32),
    mesh=_mesh,
    scratch_types=[
        pltpu.VMEM((CH,), jnp.int32),
        pltpu.VMEM((CH,), jnp.float32),
    ],
)


EB = 40
NB = CH // EB
K = 5
NG = NB // K


def _edge_body(y_hbm, scale_hbm, gidx_hbm, dst4_hbm,
               part_hbm,
               gv, dix, sbuf, msgs, shared_acc,
               semy0, semy1, semy2, semy3, semy4,
               sems0, sems1, sems2, sems3, sems4,
               semd0, semd1, semd2, semd3, semd4):
    c = lax.axis_index("c")
    s = lax.axis_index("s")
    wid = _wid()
    rows_per_tile = N2 // NS
    semy = [semy0, semy1, semy2, semy3, semy4]
    sems = [sems0, sems1, sems2, sems3, sems4]
    semd = [semd0, semd1, semd2, semd3, semd4]

    pltpu.sync_copy(gidx_hbm.at[pl.ds(wid * CH, CH)], gv)

    def zb_body(i, _):
        for j in range(4):
            for k in range(D // LANES):
                msgs[j, i, pl.ds(k * LANES, LANES)] = jnp.zeros(
                    (LANES,), jnp.float32)
        return 0
    lax.fori_loop(0, EB, zb_body, 0)
    for t in range(4):
        for j in range(4):
            pltpu.sync_copy(
                msgs.at[j],
                shared_acc.at[pl.ds(s * rows_per_tile + (t * 4 + j) * EB, EB)])
    plsc.subcore_barrier()

    def fire(b, j):
        sl = pl.ds(b * EB, EB)
        pltpu.async_copy(y_hbm.at[gv.at[sl]], msgs.at[j], semy[j])
        pltpu.async_copy(scale_hbm.at[pl.ds(wid * CH + b * EB, EB)],
                         sbuf.at[j, pl.ds(0, EB)], sems[j])
        pltpu.async_copy(dst4_hbm.at[wid, b], dix.at[j], semd[j])

    def process(b, j):
        pltpu.make_async_copy(
            y_hbm.at[pl.ds(0, EB)], msgs.at[j], semy[j]).wait()
        pltpu.make_async_copy(
            scale_hbm.at[pl.ds(0, EB)], sbuf.at[j, pl.ds(0, EB)],
            sems[j]).wait()
        pltpu.make_async_copy(dst4_hbm.at[0, 0], dix.at[j], semd[j]).wait()

        for g in range(3):
            sg = sbuf[j, pl.ds(g * LANES, LANES)]
            for l in range(LANES if g < 2 else 8):
                sv = sg[l]
                row = g * LANES + l
                for k in range(D // LANES):
                    sl = pl.ds(k * LANES, LANES)
                    msgs[j, row, sl] = msgs[j, row, sl] * sv

        pltpu.sync_copy(msgs.at[j], shared_acc.at[dix.at[j, 0]], add=True)

    for j in range(K):
        fire(j, j)

    def group_body(m, _):
        for j in range(K):
            b = m * K + j
            process(b, j)
            fire(b + K, j)
        return 0
    lax.fori_loop(0, NG - 1, group_body, 0)
    for j in range(K):
        process((NG - 1) * K + j, j)

    plsc.subcore_barrier()
    sl = pl.ds(s * rows_per_tile, rows_per_tile)
    pltpu.sync_copy(shared_acc.at[sl], part_hbm.at[c, sl])


_edge = pl.kernel(
    _edge_body,
    out_type=jax.ShapeDtypeStruct((NC, N2, D), jnp.float32),
    mesh=_mesh,
    scratch_types=[
        pltpu.VMEM((CH,), jnp.int32),
        pltpu.VMEM((K, 1, EB), jnp.int32),
        pltpu.VMEM((K, 3 * LANES), jnp.float32),
        pltpu.VMEM((K, EB, D), jnp.float32),
        pltpu.VMEM_SHARED((N2, D), jnp.float32),
    ] + [pltpu.SemaphoreType.DMA] * (3 * K),
)


def _inv_body(cnt_ref, inv_ref):
    inv_ref[...] = 1.0 / jnp.maximum(cnt_ref[0] + cnt_ref[1], 1.0)


def _tc_inv(cnt_part):
    return pl.pallas_call(
        _inv_body,
        out_shape=jax.ShapeDtypeStruct((CNT_ROWS, 128), jnp.float32),
    )(cnt_part.reshape(NC, CNT_ROWS, 128))


_BN = 1000


def _mm1_body(x_ref, wrel_ref, wroot_ref, b_ref, y_ref, root_ref):
    xb = x_ref[...]
    root_ref[...] = jnp.dot(xb, wroot_ref[...],
                            preferred_element_type=jnp.float32) + b_ref[0]
    for r in range(R):
        y_ref[r] = jnp.dot(xb, wrel_ref[r], preferred_element_type=jnp.float32)


def _tc_mm1(x, w_rel, w_root, b):
    return pl.pallas_call(
        _mm1_body,
        grid=(N // _BN,),
        in_specs=[
            pl.BlockSpec((_BN, D), lambda i: (i, 0)),
            pl.BlockSpec((R, D, D), lambda i: (0, 0, 0)),
            pl.BlockSpec((D, D), lambda i: (0, 0)),
            pl.BlockSpec((1, D), lambda i: (0, 0)),
        ],
        out_specs=[
            pl.BlockSpec((R, _BN, D), lambda i: (0, i, 0)),
            pl.BlockSpec((_BN, D), lambda i: (i, 0)),
        ],
        out_shape=[
            jax.ShapeDtypeStruct((R, N, D), jnp.float32),
            jax.ShapeDtypeStruct((N, D), jnp.float32),
        ],
    )(x, w_rel, w_root, b.reshape(1, D))


def _mm2_body(part_ref, root1_ref, wrel_ref, wroot_ref, b_ref, y_ref, root_ref):
    hb = jnp.maximum(part_ref[0] + part_ref[1] + root1_ref[...], 0.0)
    root_ref[...] = jnp.dot(hb, wroot_ref[...],
                            preferred_element_type=jnp.float32) + b_ref[0]
    for r in range(R):
        y_ref[r] = jnp.dot(hb, wrel_ref[r], preferred_element_type=jnp.float32)


def _tc_mm2(part, root1, w_rel, w_root, b):
    return pl.pallas_call(
        _mm2_body,
        grid=(N // _BN,),
        in_specs=[
            pl.BlockSpec((NC, _BN, D), lambda i: (0, i, 0)),
            pl.BlockSpec((_BN, D), lambda i: (i, 0)),
            pl.BlockSpec((R, D, D), lambda i: (0, 0, 0)),
            pl.BlockSpec((D, D), lambda i: (0, 0)),
            pl.BlockSpec((1, D), lambda i: (0, 0)),
        ],
        out_specs=[
            pl.BlockSpec((R, _BN, D), lambda i: (0, i, 0)),
            pl.BlockSpec((_BN, D), lambda i: (i, 0)),
        ],
        out_shape=[
            jax.ShapeDtypeStruct((R, N, D), jnp.float32),
            jax.ShapeDtypeStruct((N, D), jnp.float32),
        ],
    )(part, root1, w_rel, w_root, b.reshape(1, D))


def _final_body(part_ref, root_ref, out_ref):
    out_ref[...] = part_ref[0] + part_ref[1] + root_ref[...]


def _tc_final(part, root):
    return pl.pallas_call(
        _final_body,
        grid=(N // _BN,),
        in_specs=[
            pl.BlockSpec((NC, _BN, D), lambda i: (0, i, 0)),
            pl.BlockSpec((_BN, D), lambda i: (i, 0)),
        ],
        out_specs=pl.BlockSpec((_BN, D), lambda i: (i, 0)),
        out_shape=jax.ShapeDtypeStruct((N, D), jnp.float32),
    )(part, root)


def kernel(x, edge_index, edge_attr, w_rel1, w_root1, b1, w_rel2, w_root2, b2):
    src = edge_index[0]
    dst = edge_index[1]

    gidx, cidx, cnt_part = _preproc(src, dst, edge_attr)
    inv1d = _tc_inv(cnt_part).reshape(CNT)
    scale = _scale(cidx, inv1d)
    dst4 = dst.reshape(NW, NB, 1, EB)

    y1, root1 = _tc_mm1(x, w_rel1, w_root1, b1)
    part1 = _edge(y1.reshape(R * N, D), scale, gidx, dst4)

    y2, root2 = _tc_mm2(part1, root1, w_rel2, w_root2, b2)
    part2 = _edge(y2.reshape(R * N, D), scale, gidx, dst4)

    return _tc_final(part2, root2)


# speedup vs baseline: 13.4156x; 1.0024x over previous
import jax
import jax.numpy as jnp
from jax import lax
from jax.experimental import pallas as pl
from jax.experimental.pallas import tpu as pltpu
from jax.experimental.pallas import tpu_sc as plsc

NC = 2
NS = 16
NW = NC * NS
LANES = 16

N = 10000
E = 320000
D = 128
R = 3
NPAD = 10240
N2 = 10240
CNT = R * NPAD
CNT_ROWS = CNT // 128
CH = E // NW
PB = 80
PNB = CH // PB

_mesh = plsc.VectorSubcoreMesh(core_axis_name="c", subcore_axis_name="s")


def _wid():
    return lax.axis_index("s") * NC + lax.axis_index("c")


def _preproc_body(src_hbm, dst_hbm, attr_hbm,
                  gidx_hbm, cidx_hbm, cnt_hbm,
                  sv, dv, av, gv, cv, ones_v, ix_v, zb_v, shared_cnt):
    c = lax.axis_index("c")
    s = lax.axis_index("s")
    wid = _wid()
    base = wid * CH

    pltpu.sync_copy(src_hbm.at[pl.ds(base, CH)], sv)
    pltpu.sync_copy(dst_hbm.at[pl.ds(base, CH)], dv)
    pltpu.sync_copy(attr_hbm.at[pl.ds(base, CH)], av)

    def zb_body(i, _):
        zb_v[pl.ds(i * LANES, LANES)] = jnp.zeros((LANES,), jnp.float32)
        return 0
    lax.fori_loop(0, (CNT // NS) // LANES, zb_body, 0)

    for k in range(PB // LANES):
        ones_v[pl.ds(k * LANES, LANES)] = jnp.ones((LANES,), jnp.float32)

    def idx_body(i, _):
        sl = pl.ds(i * LANES, LANES)
        a = av[sl]
        gv[sl] = a * N + sv[sl]
        cv[sl] = a * NPAD + dv[sl]
        return 0
    lax.fori_loop(0, CH // LANES, idx_body, 0)

    pltpu.sync_copy(gv, gidx_hbm.at[pl.ds(base, CH)])
    pltpu.sync_copy(cv, cidx_hbm.at[pl.ds(base, CH)])

    pltpu.sync_copy(zb_v, shared_cnt.at[pl.ds(s * (CNT // NS), CNT // NS)])
    plsc.subcore_barrier()

    def cnt_body(b, _):
        off = b * PB
        for k in range(PB // LANES):
            sl = pl.ds(k * LANES, LANES)
            ix_v[sl] = cv[pl.ds(off + k * LANES, LANES)]
        pltpu.sync_copy(ones_v, shared_cnt.at[ix_v], add=True)
        return 0
    lax.fori_loop(0, PNB, cnt_body, 0)

    plsc.subcore_barrier()
    sl = pl.ds(s * (CNT // NS), CNT // NS)
    pltpu.sync_copy(shared_cnt.at[sl],
                    cnt_hbm.at[pl.ds(c * CNT + s * (CNT // NS), CNT // NS)])


_preproc = pl.kernel(
    _preproc_body,
    out_type=(
        jax.ShapeDtypeStruct((E,), jnp.int32),
        jax.ShapeDtypeStruct((E,), jnp.int32),
        jax.ShapeDtypeStruct((NC * CNT,), jnp.float32),
    ),
    mesh=_mesh,
    scratch_types=[
        pltpu.VMEM((CH,), jnp.int32),
        pltpu.VMEM((CH,), jnp.int32),
        pltpu.VMEM((CH,), jnp.int32),
        pltpu.VMEM((CH,), jnp.int32),
        pltpu.VMEM((CH,), jnp.int32),
        pltpu.VMEM((PB,), jnp.float32),
        pltpu.VMEM((PB,), jnp.int32),
        pltpu.VMEM((CNT // NS,), jnp.float32),
        pltpu.VMEM_SHARED((CNT,), jnp.float32),
    ],
)


SB = 80


def _scale_body(cidx_hbm, inv_hbm, scale_hbm, cv, sbig):
    wid = _wid()
    base = wid * CH
    pltpu.sync_copy(cidx_hbm.at[pl.ds(base, CH)], cv)

    def b_body(b, _):
        sl = pl.ds(b * SB, SB)
        pltpu.sync_copy(inv_hbm.at[cv.at[sl]], sbig.at[sl])
        return 0
    lax.fori_loop(0, CH // SB, b_body, 0)
    pltpu.sync_copy(sbig, scale_hbm.at[pl.ds(base, CH)])


_scale = pl.kernel(
    _scale_body,
    out_type=jax.ShapeDtypeStruct((E,), jnp.float32),
    mesh=_mesh,
    scratch_types=[
        pltpu.VMEM((CH,), jnp.int32),
        pltpu.VMEM((CH,), jnp.float32),
    ],
)


EB = 40
NB = CH // EB
K = 5
NG = NB // K


def _edge_body(y_hbm, scale_hbm, gidx_hbm, dst4_hbm,
               part_hbm,
               gv, dix, sbuf, msgs, shared_acc,
               semy0, semy1, semy2, semy3, semy4,
               sems0, sems1, sems2, sems3, sems4,
               semd0, semd1, semd2, semd3, semd4,
               semc0, semc1, semc2, semc3, semc4):
    c = lax.axis_index("c")
    s = lax.axis_index("s")
    wid = _wid()
    rows_per_tile = N2 // NS
    semy = [semy0, semy1, semy2, semy3, semy4]
    sems = [sems0, sems1, sems2, sems3, sems4]
    semd = [semd0, semd1, semd2, semd3, semd4]
    semc = [semc0, semc1, semc2, semc3, semc4]

    pltpu.sync_copy(gidx_hbm.at[pl.ds(wid * CH, CH)], gv)

    def zb_body(i, _):
        for j in range(4):
            for k in range(D // LANES):
                msgs[j, i, pl.ds(k * LANES, LANES)] = jnp.zeros(
                    (LANES,), jnp.float32)
        return 0
    lax.fori_loop(0, EB, zb_body, 0)
    for t in range(4):
        for j in range(4):
            pltpu.sync_copy(
                msgs.at[j],
                shared_acc.at[pl.ds(s * rows_per_tile + (t * 4 + j) * EB, EB)])
    plsc.subcore_barrier()

    def fire(b, j, drain_scatter=True):
        if drain_scatter:
            pltpu.make_async_copy(
                msgs.at[j], shared_acc.at[pl.ds(0, EB)], semc[j]).wait()
        sl = pl.ds(b * EB, EB)
        pltpu.async_copy(y_hbm.at[gv.at[sl]], msgs.at[j], semy[j])
        pltpu.async_copy(scale_hbm.at[pl.ds(wid * CH + b * EB, EB)],
                         sbuf.at[j, pl.ds(0, EB)], sems[j])
        pltpu.async_copy(dst4_hbm.at[wid, b], dix.at[j], semd[j])

    def process(b, j):
        pltpu.make_async_copy(
            y_hbm.at[pl.ds(0, EB)], msgs.at[j], semy[j]).wait()
        pltpu.make_async_copy(
            scale_hbm.at[pl.ds(0, EB)], sbuf.at[j, pl.ds(0, EB)],
            sems[j]).wait()
        pltpu.make_async_copy(dst4_hbm.at[0, 0], dix.at[j], semd[j]).wait()

        for g in range(3):
            sg = sbuf[j, pl.ds(g * LANES, LANES)]
            for l in range(LANES if g < 2 else 8):
                sv = sg[l]
                row = g * LANES + l
                for k in range(D // LANES):
                    sl = pl.ds(k * LANES, LANES)
                    msgs[j, row, sl] = msgs[j, row, sl] * sv

        pltpu.async_copy(msgs.at[j], shared_acc.at[dix.at[j, 0]], semc[j],
                         add=True)

    for j in range(K):
        fire(j, j, drain_scatter=False)

    def group_body(m, _):
        for j in range(K):
            b = m * K + j
            process(b, j)
            fire(b + K, j)
        return 0
    lax.fori_loop(0, NG - 1, group_body, 0)
    for j in range(K):
        process((NG - 1) * K + j, j)
    for j in range(K):
        pltpu.make_async_copy(
            msgs.at[j], shared_acc.at[pl.ds(0, EB)], semc[j]).wait()

    plsc.subcore_barrier()
    sl = pl.ds(s * rows_per_tile, rows_per_tile)
    pltpu.sync_copy(shared_acc.at[sl], part_hbm.at[c, sl])


_edge = pl.kernel(
    _edge_body,
    out_type=jax.ShapeDtypeStruct((NC, N2, D), jnp.float32),
    mesh=_mesh,
    scratch_types=[
        pltpu.VMEM((CH,), jnp.int32),
        pltpu.VMEM((K, 1, EB), jnp.int32),
        pltpu.VMEM((K, 3 * LANES), jnp.float32),
        pltpu.VMEM((K, EB, D), jnp.float32),
        pltpu.VMEM_SHARED((N2, D), jnp.float32),
    ] + [pltpu.SemaphoreType.DMA] * (4 * K),
)


def _inv_body(cnt_ref, inv_ref):
    inv_ref[...] = 1.0 / jnp.maximum(cnt_ref[0] + cnt_ref[1], 1.0)


def _tc_inv(cnt_part):
    return pl.pallas_call(
        _inv_body,
        out_shape=jax.ShapeDtypeStruct((CNT_ROWS, 128), jnp.float32),
    )(cnt_part.reshape(NC, CNT_ROWS, 128))


_BN = 1000


def _mm1_body(x_ref, wrel_ref, wroot_ref, b_ref, y_ref, root_ref):
    xb = x_ref[...]
    root_ref[...] = jnp.dot(xb, wroot_ref[...],
                            preferred_element_type=jnp.float32) + b_ref[0]
    for r in range(R):
        y_ref[r] = jnp.dot(xb, wrel_ref[r], preferred_element_type=jnp.float32)


def _tc_mm1(x, w_rel, w_root, b):
    return pl.pallas_call(
        _mm1_body,
        grid=(N // _BN,),
        in_specs=[
            pl.BlockSpec((_BN, D), lambda i: (i, 0)),
            pl.BlockSpec((R, D, D), lambda i: (0, 0, 0)),
            pl.BlockSpec((D, D), lambda i: (0, 0)),
            pl.BlockSpec((1, D), lambda i: (0, 0)),
        ],
        out_specs=[
            pl.BlockSpec((R, _BN, D), lambda i: (0, i, 0)),
            pl.BlockSpec((_BN, D), lambda i: (i, 0)),
        ],
        out_shape=[
            jax.ShapeDtypeStruct((R, N, D), jnp.float32),
            jax.ShapeDtypeStruct((N, D), jnp.float32),
        ],
    )(x, w_rel, w_root, b.reshape(1, D))


def _mm2_body(part_ref, root1_ref, wrel_ref, wroot_ref, b_ref, y_ref, root_ref):
    hb = jnp.maximum(part_ref[0] + part_ref[1] + root1_ref[...], 0.0)
    root_ref[...] = jnp.dot(hb, wroot_ref[...],
                            preferred_element_type=jnp.float32) + b_ref[0]
    for r in range(R):
        y_ref[r] = jnp.dot(hb, wrel_ref[r], preferred_element_type=jnp.float32)


def _tc_mm2(part, root1, w_rel, w_root, b):
    return pl.pallas_call(
        _mm2_body,
        grid=(N // _BN,),
        in_specs=[
            pl.BlockSpec((NC, _BN, D), lambda i: (0, i, 0)),
            pl.BlockSpec((_BN, D), lambda i: (i, 0)),
            pl.BlockSpec((R, D, D), lambda i: (0, 0, 0)),
            pl.BlockSpec((D, D), lambda i: (0, 0)),
            pl.BlockSpec((1, D), lambda i: (0, 0)),
        ],
        out_specs=[
            pl.BlockSpec((R, _BN, D), lambda i: (0, i, 0)),
            pl.BlockSpec((_BN, D), lambda i: (i, 0)),
        ],
        out_shape=[
            jax.ShapeDtypeStruct((R, N, D), jnp.float32),
            jax.ShapeDtypeStruct((N, D), jnp.float32),
        ],
    )(part, root1, w_rel, w_root, b.reshape(1, D))


def _final_body(part_ref, root_ref, out_ref):
    out_ref[...] = part_ref[0] + part_ref[1] + root_ref[...]


def _tc_final(part, root):
    return pl.pallas_call(
        _final_body,
        grid=(N // _BN,),
        in_specs=[
            pl.BlockSpec((NC, _BN, D), lambda i: (0, i, 0)),
            pl.BlockSpec((_BN, D), lambda i: (i, 0)),
        ],
        out_specs=pl.BlockSpec((_BN, D), lambda i: (i, 0)),
        out_shape=jax.ShapeDtypeStruct((N, D), jnp.float32),
    )(part, root)


def kernel(x, edge_index, edge_attr, w_rel1, w_root1, b1, w_rel2, w_root2, b2):
    src = edge_index[0]
    dst = edge_index[1]

    gidx, cidx, cnt_part = _preproc(src, dst, edge_attr)
    inv1d = _tc_inv(cnt_part).reshape(CNT)
    scale = _scale(cidx, inv1d)
    dst4 = dst.reshape(NW, NB, 1, EB)

    y1, root1 = _tc_mm1(x, w_rel1, w_root1, b1)
    part1 = _edge(y1.reshape(R * N, D), scale, gidx, dst4)

    y2, root2 = _tc_mm2(part1, root1, w_rel2, w_root2, b2)
    part2 = _edge(y2.reshape(R * N, D), scale, gidx, dst4)

    return _tc_final(part2, root2)


# speedup vs baseline: 16.3050x; 1.2154x over previous
import jax
import jax.numpy as jnp
from jax import lax
from jax.experimental import pallas as pl
from jax.experimental.pallas import tpu as pltpu
from jax.experimental.pallas import tpu_sc as plsc

NC = 2
NS = 16
NW = NC * NS
LANES = 16

N = 10000
E = 320000
D = 128
R = 3
NPAD = 10240
N2 = 10240
CNT = R * NPAD
CNT_ROWS = CNT // 128
CH = E // NW
PB = 80
PNB = CH // PB

_mesh = plsc.VectorSubcoreMesh(core_axis_name="c", subcore_axis_name="s")


def _wid():
    return lax.axis_index("s") * NC + lax.axis_index("c")


def _preproc_body(src_hbm, dst_hbm, attr_hbm,
                  gidx_hbm, cidx_hbm, cnt_hbm,
                  sv, dv, av, gv, cv, ones_v, ix_v, zb_v, shared_cnt):
    c = lax.axis_index("c")
    s = lax.axis_index("s")
    wid = _wid()
    base = wid * CH

    pltpu.sync_copy(src_hbm.at[pl.ds(base, CH)], sv)
    pltpu.sync_copy(dst_hbm.at[pl.ds(base, CH)], dv)
    pltpu.sync_copy(attr_hbm.at[pl.ds(base, CH)], av)

    def zb_body(i, _):
        zb_v[pl.ds(i * LANES, LANES)] = jnp.zeros((LANES,), jnp.float32)
        return 0
    lax.fori_loop(0, (CNT // NS) // LANES, zb_body, 0)

    for k in range(PB // LANES):
        ones_v[pl.ds(k * LANES, LANES)] = jnp.ones((LANES,), jnp.float32)

    def idx_body(i, _):
        sl = pl.ds(i * LANES, LANES)
        a = av[sl]
        gv[sl] = a * N + sv[sl]
        cv[sl] = a * NPAD + dv[sl]
        return 0
    lax.fori_loop(0, CH // LANES, idx_body, 0)

    pltpu.sync_copy(gv, gidx_hbm.at[pl.ds(base, CH)])
    pltpu.sync_copy(cv, cidx_hbm.at[pl.ds(base, CH)])

    pltpu.sync_copy(zb_v, shared_cnt.at[pl.ds(s * (CNT // NS), CNT // NS)])
    plsc.subcore_barrier()

    def cnt_body(b, _):
        off = b * PB
        for k in range(PB // LANES):
            sl = pl.ds(k * LANES, LANES)
            ix_v[sl] = cv[pl.ds(off + k * LANES, LANES)]
        pltpu.sync_copy(ones_v, shared_cnt.at[ix_v], add=True)
        return 0
    lax.fori_loop(0, PNB, cnt_body, 0)

    plsc.subcore_barrier()
    sl = pl.ds(s * (CNT // NS), CNT // NS)
    pltpu.sync_copy(shared_cnt.at[sl],
                    cnt_hbm.at[pl.ds(c * CNT + s * (CNT // NS), CNT // NS)])


_preproc = pl.kernel(
    _preproc_body,
    out_type=(
        jax.ShapeDtypeStruct((E,), jnp.int32),
        jax.ShapeDtypeStruct((E,), jnp.int32),
        jax.ShapeDtypeStruct((NC * CNT,), jnp.float32),
    ),
    mesh=_mesh,
    scratch_types=[
        pltpu.VMEM((CH,), jnp.int32),
        pltpu.VMEM((CH,), jnp.int32),
        pltpu.VMEM((CH,), jnp.int32),
        pltpu.VMEM((CH,), jnp.int32),
        pltpu.VMEM((CH,), jnp.int32),
        pltpu.VMEM((PB,), jnp.float32),
        pltpu.VMEM((PB,), jnp.int32),
        pltpu.VMEM((CNT // NS,), jnp.float32),
        pltpu.VMEM_SHARED((CNT,), jnp.float32),
    ],
)


EB = 40
NB = CH // EB


def _edge_body(y_hbm, inv_hbm, gidx_hbm, cidx_hbm, dst4_hbm,
               part_hbm,
               gv, cv, dix, sbuf, msgs, shared_acc,
               semy0, semy1, sems0, sems1, semd0, semd1):
    c = lax.axis_index("c")
    s = lax.axis_index("s")
    wid = _wid()
    rows_per_tile = N2 // NS
    semy = [semy0, semy1]
    sems = [sems0, sems1]
    semd = [semd0, semd1]

    pltpu.sync_copy(gidx_hbm.at[pl.ds(wid * CH, CH)], gv)
    pltpu.sync_copy(cidx_hbm.at[pl.ds(wid * CH, CH)], cv)

    def zb_body(i, _):
        for j in range(2):
            for k in range(D // LANES):
                msgs[j, i, pl.ds(k * LANES, LANES)] = jnp.zeros(
                    (LANES,), jnp.float32)
        return 0
    lax.fori_loop(0, EB, zb_body, 0)
    for t in range(16):
        pltpu.sync_copy(
            msgs.at[t % 2],
            shared_acc.at[pl.ds(s * rows_per_tile + t * EB, EB)])
    plsc.subcore_barrier()

    def fire(b, j):
        sl = pl.ds(b * EB, EB)
        pltpu.async_copy(y_hbm.at[gv.at[sl]], msgs.at[j], semy[j])
        pltpu.async_copy(inv_hbm.at[cv.at[sl]], sbuf.at[j, pl.ds(0, EB)],
                         sems[j])
        pltpu.async_copy(dst4_hbm.at[wid, b], dix.at[j], semd[j])

    def process(b, j):
        pltpu.make_async_copy(
            y_hbm.at[pl.ds(0, EB)], msgs.at[j], semy[j]).wait()
        pltpu.make_async_copy(
            inv_hbm.at[pl.ds(0, EB)], sbuf.at[j, pl.ds(0, EB)], sems[j]).wait()
        pltpu.make_async_copy(dst4_hbm.at[0, 0], dix.at[j], semd[j]).wait()

        for g in range(3):
            sg = sbuf[j, pl.ds(g * LANES, LANES)]
            for l in range(LANES if g < 2 else 8):
                sv = sg[l]
                row = g * LANES + l
                for k in range(D // LANES):
                    sl = pl.ds(k * LANES, LANES)
                    msgs[j, row, sl] = msgs[j, row, sl] * sv

        pltpu.sync_copy(msgs.at[j], shared_acc.at[dix.at[j, 0]], add=True)

    fire(0, 0)
    fire(1, 1)

    def pair_body(q, _):
        b = 2 * q
        process(b, 0)
        fire(b + 2, 0)
        process(b + 1, 1)
        fire(b + 3, 1)
        return 0
    lax.fori_loop(0, NB // 2 - 1, pair_body, 0)
    process(NB - 2, 0)
    process(NB - 1, 1)

    plsc.subcore_barrier()
    sl = pl.ds(s * rows_per_tile, rows_per_tile)
    pltpu.sync_copy(shared_acc.at[sl], part_hbm.at[c, sl])


_edge = pl.kernel(
    _edge_body,
    out_type=jax.ShapeDtypeStruct((NC, N2, D), jnp.float32),
    mesh=_mesh,
    scratch_types=[
        pltpu.VMEM((CH,), jnp.int32),
        pltpu.VMEM((CH,), jnp.int32),
        pltpu.VMEM((2, 1, EB), jnp.int32),
        pltpu.VMEM((2, 3 * LANES), jnp.float32),
        pltpu.VMEM((2, EB, D), jnp.float32),
        pltpu.VMEM_SHARED((N2, D), jnp.float32),
    ] + [pltpu.SemaphoreType.DMA] * 6,
)


def _inv_body(cnt_ref, inv_ref):
    inv_ref[...] = 1.0 / jnp.maximum(cnt_ref[0] + cnt_ref[1], 1.0)


def _tc_inv(cnt_part):
    return pl.pallas_call(
        _inv_body,
        out_shape=jax.ShapeDtypeStruct((CNT_ROWS, 128), jnp.float32),
    )(cnt_part.reshape(NC, CNT_ROWS, 128))


_BN = 1000


def _mm1_body(x_ref, wrel_ref, wroot_ref, b_ref, y_ref, root_ref):
    xb = x_ref[...]
    root_ref[...] = jnp.dot(xb, wroot_ref[...],
                            preferred_element_type=jnp.float32) + b_ref[0]
    for r in range(R):
        y_ref[r] = jnp.dot(xb, wrel_ref[r], preferred_element_type=jnp.float32)


def _tc_mm1(x, w_rel, w_root, b):
    return pl.pallas_call(
        _mm1_body,
        grid=(N // _BN,),
        in_specs=[
            pl.BlockSpec((_BN, D), lambda i: (i, 0)),
            pl.BlockSpec((R, D, D), lambda i: (0, 0, 0)),
            pl.BlockSpec((D, D), lambda i: (0, 0)),
            pl.BlockSpec((1, D), lambda i: (0, 0)),
        ],
        out_specs=[
            pl.BlockSpec((R, _BN, D), lambda i: (0, i, 0)),
            pl.BlockSpec((_BN, D), lambda i: (i, 0)),
        ],
        out_shape=[
            jax.ShapeDtypeStruct((R, N, D), jnp.float32),
            jax.ShapeDtypeStruct((N, D), jnp.float32),
        ],
    )(x, w_rel, w_root, b.reshape(1, D))


def _mm2_body(part_ref, root1_ref, wrel_ref, wroot_ref, b_ref, y_ref, root_ref):
    hb = jnp.maximum(part_ref[0] + part_ref[1] + root1_ref[...], 0.0)
    root_ref[...] = jnp.dot(hb, wroot_ref[...],
                            preferred_element_type=jnp.float32) + b_ref[0]
    for r in range(R):
        y_ref[r] = jnp.dot(hb, wrel_ref[r], preferred_element_type=jnp.float32)


def _tc_mm2(part, root1, w_rel, w_root, b):
    return pl.pallas_call(
        _mm2_body,
        grid=(N // _BN,),
        in_specs=[
            pl.BlockSpec((NC, _BN, D), lambda i: (0, i, 0)),
            pl.BlockSpec((_BN, D), lambda i: (i, 0)),
            pl.BlockSpec((R, D, D), lambda i: (0, 0, 0)),
            pl.BlockSpec((D, D), lambda i: (0, 0)),
            pl.BlockSpec((1, D), lambda i: (0, 0)),
        ],
        out_specs=[
            pl.BlockSpec((R, _BN, D), lambda i: (0, i, 0)),
            pl.BlockSpec((_BN, D), lambda i: (i, 0)),
        ],
        out_shape=[
            jax.ShapeDtypeStruct((R, N, D), jnp.float32),
            jax.ShapeDtypeStruct((N, D), jnp.float32),
        ],
    )(part, root1, w_rel, w_root, b.reshape(1, D))


def _final_body(part_ref, root_ref, out_ref):
    out_ref[...] = part_ref[0] + part_ref[1] + root_ref[...]


def _tc_final(part, root):
    return pl.pallas_call(
        _final_body,
        grid=(N // _BN,),
        in_specs=[
            pl.BlockSpec((NC, _BN, D), lambda i: (0, i, 0)),
            pl.BlockSpec((_BN, D), lambda i: (i, 0)),
        ],
        out_specs=pl.BlockSpec((_BN, D), lambda i: (i, 0)),
        out_shape=jax.ShapeDtypeStruct((N, D), jnp.float32),
    )(part, root)


def kernel(x, edge_index, edge_attr, w_rel1, w_root1, b1, w_rel2, w_root2, b2):
    src = edge_index[0]
    dst = edge_index[1]

    gidx, cidx, cnt_part = _preproc(src, dst, edge_attr)
    inv1d = _tc_inv(cnt_part).reshape(CNT)
    dst4 = dst.reshape(NW, NB, 1, EB)

    y1, root1 = _tc_mm1(x, w_rel1, w_root1, b1)
    part1 = _edge(y1.reshape(R * N, D), inv1d, gidx, cidx, dst4)

    y2, root2 = _tc_mm2(part1, root1, w_rel2, w_root2, b2)
    part2 = _edge(y2.reshape(R * N, D), inv1d, gidx, cidx, dst4)

    return _tc_final(part2, root2)


# speedup vs baseline: 19.3069x; 1.1841x over previous
import jax
import jax.numpy as jnp
from jax import lax
from jax.experimental import pallas as pl
from jax.experimental.pallas import tpu as pltpu
from jax.experimental.pallas import tpu_sc as plsc

NC = 2
NS = 16
NW = NC * NS
LANES = 16

N = 10000
E = 320000
D = 128
R = 3
NPAD = 10240
N2 = 10240
CNT = R * NPAD
CNT_ROWS = CNT // 128
CH = E // NW
PB = 80
PNB = CH // PB

_mesh = plsc.VectorSubcoreMesh(core_axis_name="c", subcore_axis_name="s")


def _wid():
    return lax.axis_index("s") * NC + lax.axis_index("c")


def _preproc_body(src_hbm, dst_hbm, attr_hbm,
                  gidx_hbm, cidx_hbm, cnt_hbm,
                  sv, dv, av, gv, cv, ones_v, ix_v, zb_v, shared_cnt):
    c = lax.axis_index("c")
    s = lax.axis_index("s")
    wid = _wid()
    base = wid * CH

    pltpu.sync_copy(src_hbm.at[pl.ds(base, CH)], sv)
    pltpu.sync_copy(dst_hbm.at[pl.ds(base, CH)], dv)
    pltpu.sync_copy(attr_hbm.at[pl.ds(base, CH)], av)

    def zb_body(i, _):
        zb_v[pl.ds(i * LANES, LANES)] = jnp.zeros((LANES,), jnp.float32)
        return 0
    lax.fori_loop(0, (CNT // NS) // LANES, zb_body, 0)

    for k in range(PB // LANES):
        ones_v[pl.ds(k * LANES, LANES)] = jnp.ones((LANES,), jnp.float32)

    def idx_body(i, _):
        sl = pl.ds(i * LANES, LANES)
        a = av[sl]
        gv[sl] = a * N + sv[sl]
        cv[sl] = a * NPAD + dv[sl]
        return 0
    lax.fori_loop(0, CH // LANES, idx_body, 0)

    pltpu.sync_copy(gv, gidx_hbm.at[pl.ds(base, CH)])
    pltpu.sync_copy(cv, cidx_hbm.at[pl.ds(base, CH)])

    pltpu.sync_copy(zb_v, shared_cnt.at[pl.ds(s * (CNT // NS), CNT // NS)])
    plsc.subcore_barrier()

    def cnt_body(b, _):
        off = b * PB
        for k in range(PB // LANES):
            sl = pl.ds(k * LANES, LANES)
            ix_v[sl] = cv[pl.ds(off + k * LANES, LANES)]
        pltpu.sync_copy(ones_v, shared_cnt.at[ix_v], add=True)
        return 0
    lax.fori_loop(0, PNB, cnt_body, 0)

    plsc.subcore_barrier()
    sl = pl.ds(s * (CNT // NS), CNT // NS)
    pltpu.sync_copy(shared_cnt.at[sl],
                    cnt_hbm.at[pl.ds(c * CNT + s * (CNT // NS), CNT // NS)])


_preproc = pl.kernel(
    _preproc_body,
    out_type=(
        jax.ShapeDtypeStruct((E,), jnp.int32),
        jax.ShapeDtypeStruct((E,), jnp.int32),
        jax.ShapeDtypeStruct((NC * CNT,), jnp.float32),
    ),
    mesh=_mesh,
    scratch_types=[
        pltpu.VMEM((CH,), jnp.int32),
        pltpu.VMEM((CH,), jnp.int32),
        pltpu.VMEM((CH,), jnp.int32),
        pltpu.VMEM((CH,), jnp.int32),
        pltpu.VMEM((CH,), jnp.int32),
        pltpu.VMEM((PB,), jnp.float32),
        pltpu.VMEM((PB,), jnp.int32),
        pltpu.VMEM((CNT // NS,), jnp.float32),
        pltpu.VMEM_SHARED((CNT,), jnp.float32),
    ],
)


EB = 80
NB = CH // EB


def _edge_body(y_hbm, inv_hbm, gidx_hbm, cidx_hbm, dst4_hbm,
               part_hbm,
               gv, cv, dix, sbuf, msgs, shared_acc,
               semy0, semy1, sems0, sems1, semd0, semd1):
    c = lax.axis_index("c")
    s = lax.axis_index("s")
    wid = _wid()
    rows_per_tile = N2 // NS
    semy = [semy0, semy1]
    sems = [sems0, sems1]
    semd = [semd0, semd1]

    pltpu.sync_copy(gidx_hbm.at[pl.ds(wid * CH, CH)], gv)
    pltpu.sync_copy(cidx_hbm.at[pl.ds(wid * CH, CH)], cv)

    def zb_body(i, _):
        for j in range(2):
            for k in range(D // LANES):
                msgs[j, i, pl.ds(k * LANES, LANES)] = jnp.zeros(
                    (LANES,), jnp.float32)
        return 0
    lax.fori_loop(0, EB, zb_body, 0)
    for t in range(8):
        pltpu.sync_copy(
            msgs.at[t % 2],
            shared_acc.at[pl.ds(s * rows_per_tile + t * EB, EB)])
    plsc.subcore_barrier()

    def fire(b, j):
        sl = pl.ds(b * EB, EB)
        pltpu.async_copy(y_hbm.at[gv.at[sl]], msgs.at[j], semy[j])
        pltpu.async_copy(inv_hbm.at[cv.at[sl]], sbuf.at[j, pl.ds(0, EB)],
                         sems[j])
        pltpu.async_copy(dst4_hbm.at[wid, b], dix.at[j], semd[j])

    def process(b, j):
        pltpu.make_async_copy(
            y_hbm.at[pl.ds(0, EB)], msgs.at[j], semy[j]).wait()
        pltpu.make_async_copy(
            inv_hbm.at[pl.ds(0, EB)], sbuf.at[j, pl.ds(0, EB)], sems[j]).wait()
        pltpu.make_async_copy(dst4_hbm.at[0, 0], dix.at[j], semd[j]).wait()

        for g in range(EB // LANES):
            sg = sbuf[j, pl.ds(g * LANES, LANES)]
            for l in range(LANES):
                sv = sg[l]
                row = g * LANES + l
                for k in range(D // LANES):
                    sl = pl.ds(k * LANES, LANES)
                    msgs[j, row, sl] = msgs[j, row, sl] * sv

        pltpu.sync_copy(msgs.at[j], shared_acc.at[dix.at[j, 0]], add=True)

    fire(0, 0)
    fire(1, 1)

    def pair_body(q, _):
        b = 2 * q
        process(b, 0)
        fire(b + 2, 0)
        process(b + 1, 1)
        fire(b + 3, 1)
        return 0
    lax.fori_loop(0, NB // 2 - 1, pair_body, 0)
    process(NB - 3, 0)
    fire(NB - 1, 0)
    process(NB - 2, 1)
    process(NB - 1, 0)

    plsc.subcore_barrier()
    sl = pl.ds(s * rows_per_tile, rows_per_tile)
    pltpu.sync_copy(shared_acc.at[sl], part_hbm.at[c, sl])


_edge = pl.kernel(
    _edge_body,
    out_type=jax.ShapeDtypeStruct((NC, N2, D), jnp.float32),
    mesh=_mesh,
    scratch_types=[
        pltpu.VMEM((CH,), jnp.int32),
        pltpu.VMEM((CH,), jnp.int32),
        pltpu.VMEM((2, 1, EB), jnp.int32),
        pltpu.VMEM((2, EB), jnp.float32),
        pltpu.VMEM((2, EB, D), jnp.float32),
        pltpu.VMEM_SHARED((N2, D), jnp.float32),
    ] + [pltpu.SemaphoreType.DMA] * 6,
)


def _inv_body(cnt_ref, inv_ref):
    inv_ref[...] = 1.0 / jnp.maximum(cnt_ref[0] + cnt_ref[1], 1.0)


def _tc_inv(cnt_part):
    return pl.pallas_call(
        _inv_body,
        out_shape=jax.ShapeDtypeStruct((CNT_ROWS, 128), jnp.float32),
    )(cnt_part.reshape(NC, CNT_ROWS, 128))


_BN = 1000


def _mm1_body(x_ref, wrel_ref, wroot_ref, b_ref, y_ref, root_ref):
    xb = x_ref[...]
    root_ref[...] = jnp.dot(xb, wroot_ref[...],
                            preferred_element_type=jnp.float32) + b_ref[0]
    for r in range(R):
        y_ref[r] = jnp.dot(xb, wrel_ref[r], preferred_element_type=jnp.float32)


def _tc_mm1(x, w_rel, w_root, b):
    return pl.pallas_call(
        _mm1_body,
        grid=(N // _BN,),
        in_specs=[
            pl.BlockSpec((_BN, D), lambda i: (i, 0)),
            pl.BlockSpec((R, D, D), lambda i: (0, 0, 0)),
            pl.BlockSpec((D, D), lambda i: (0, 0)),
            pl.BlockSpec((1, D), lambda i: (0, 0)),
        ],
        out_specs=[
            pl.BlockSpec((R, _BN, D), lambda i: (0, i, 0)),
            pl.BlockSpec((_BN, D), lambda i: (i, 0)),
        ],
        out_shape=[
            jax.ShapeDtypeStruct((R, N, D), jnp.float32),
            jax.ShapeDtypeStruct((N, D), jnp.float32),
        ],
    )(x, w_rel, w_root, b.reshape(1, D))


def _mm2_body(part_ref, root1_ref, wrel_ref, wroot_ref, b_ref, y_ref, root_ref):
    hb = jnp.maximum(part_ref[0] + part_ref[1] + root1_ref[...], 0.0)
    root_ref[...] = jnp.dot(hb, wroot_ref[...],
                            preferred_element_type=jnp.float32) + b_ref[0]
    for r in range(R):
        y_ref[r] = jnp.dot(hb, wrel_ref[r], preferred_element_type=jnp.float32)


def _tc_mm2(part, root1, w_rel, w_root, b):
    return pl.pallas_call(
        _mm2_body,
        grid=(N // _BN,),
        in_specs=[
            pl.BlockSpec((NC, _BN, D), lambda i: (0, i, 0)),
            pl.BlockSpec((_BN, D), lambda i: (i, 0)),
            pl.BlockSpec((R, D, D), lambda i: (0, 0, 0)),
            pl.BlockSpec((D, D), lambda i: (0, 0)),
            pl.BlockSpec((1, D), lambda i: (0, 0)),
        ],
        out_specs=[
            pl.BlockSpec((R, _BN, D), lambda i: (0, i, 0)),
            pl.BlockSpec((_BN, D), lambda i: (i, 0)),
        ],
        out_shape=[
            jax.ShapeDtypeStruct((R, N, D), jnp.float32),
            jax.ShapeDtypeStruct((N, D), jnp.float32),
        ],
    )(part, root1, w_rel, w_root, b.reshape(1, D))


def _final_body(part_ref, root_ref, out_ref):
    out_ref[...] = part_ref[0] + part_ref[1] + root_ref[...]


def _tc_final(part, root):
    return pl.pallas_call(
        _final_body,
        grid=(N // _BN,),
        in_specs=[
            pl.BlockSpec((NC, _BN, D), lambda i: (0, i, 0)),
            pl.BlockSpec((_BN, D), lambda i: (i, 0)),
        ],
        out_specs=pl.BlockSpec((_BN, D), lambda i: (i, 0)),
        out_shape=jax.ShapeDtypeStruct((N, D), jnp.float32),
    )(part, root)


def kernel(x, edge_index, edge_attr, w_rel1, w_root1, b1, w_rel2, w_root2, b2):
    src = edge_index[0]
    dst = edge_index[1]

    gidx, cidx, cnt_part = _preproc(src, dst, edge_attr)
    inv1d = _tc_inv(cnt_part).reshape(CNT)
    dst4 = dst.reshape(NW, NB, 1, EB)

    y1, root1 = _tc_mm1(x, w_rel1, w_root1, b1)
    part1 = _edge(y1.reshape(R * N, D), inv1d, gidx, cidx, dst4)

    y2, root2 = _tc_mm2(part1, root1, w_rel2, w_root2, b2)
    part2 = _edge(y2.reshape(R * N, D), inv1d, gidx, cidx, dst4)

    return _tc_final(part2, root2)
